# baseline (speedup 1.0000x reference)
import functools

import jax
import jax.numpy as jnp
from jax import lax
from jax.experimental import pallas as pl
from jax.experimental.pallas import tpu as pltpu
from jax.experimental.pallas import tpu_sc as plsc

VOCAB = 100000
ES = 32
HS = 64
B = 4096
T = 50

_TC = 10
_NCHUNK = T // _TC

_NC = 2
_NS = 16
_NW = _NC * _NS
_CROWS = B * _TC
_RPW = _CROWS // _NW
_GRP = 128
_NGRP = _RPW // _GRP


def _sc_gather_body(emb_hbm, idx_hbm, out_hbm, idx_v, rows_v, sem):
  wid = lax.axis_index("s") * _NC + lax.axis_index("c")
  pltpu.sync_copy(idx_hbm.at[wid], idx_v)
  copies = []
  for j in range(_NGRP):
    cp = pltpu.async_copy(emb_hbm.at[idx_v.at[j]], rows_v.at[j], sem)
    copies.append(cp)
  for cp in copies:
    cp.wait()
  pltpu.sync_copy(rows_v, out_hbm.at[pl.ds(wid * _NGRP, _NGRP)])


@functools.cache
def _sc_gather():
  return functools.partial(
      pl.kernel,
      out_type=jax.ShapeDtypeStruct((_CROWS // _GRP, _GRP, ES), jnp.float32),
      mesh=plsc.VectorSubcoreMesh(core_axis_name="c", subcore_axis_name="s"),
      scratch_types=[
          pltpu.VMEM((_NGRP, _GRP), jnp.int32),
          pltpu.VMEM((_NGRP, _GRP, ES), jnp.float32),
          pltpu.SemaphoreType.DMA,
      ],
      compiler_params=pltpu.CompilerParams(use_tc_tiling_on_sc=False),
  )(_sc_gather_body)


_F = 4
_FH = _F * HS
_FE = _F * ES
_BQ = B // _F


def _lfold_body(xt_ref, ones_ref, q_ref, out_ref):
  ind = (xt_ref[...] != 0).astype(jnp.float32)
  lcol = lax.dot_general(ind, ones_ref[...], (((0,), (0,)), ((), ())),
                         preferred_element_type=jnp.float32)
  out_ref[...] = jnp.dot(lcol.reshape(_BQ, _F), q_ref[...],
                         preferred_element_type=jnp.float32).astype(jnp.int32)


def _lfold(xt, ones_t, qmat):
  return pl.pallas_call(
      _lfold_body,
      in_specs=[
          pl.BlockSpec((T, B), lambda: (0, 0)),
          pl.BlockSpec((T, 1), lambda: (0, 0)),
          pl.BlockSpec((_F, _FH), lambda: (0, 0)),
      ],
      out_specs=pl.BlockSpec((_BQ, _FH), lambda: (0, 0)),
      out_shape=jax.ShapeDtypeStruct((_BQ, _FH), jnp.int32),
  )(xt, ones_t, qmat)


def _gru_chunk_body(t0, e_ref, lf_ref, h_ref, wi_ref, wh_ref, gib_ref,
                    bhn_ref, out_ref, gi_s):
  e2d = e_ref[...].reshape(_TC * _BQ, _FE)
  gi_s[...] = (jnp.dot(e2d, wi_ref[...], preferred_element_type=jnp.float32)
               + gib_ref[...]).reshape(_TC, _BQ, 3 * _FH)
  wh = wh_ref[...]
  bhn = bhn_ref[...]
  lfold = lf_ref[...]

  def step(k, h):
    gi = gi_s[k]
    gh = jnp.dot(h, wh, preferred_element_type=jnp.float32)
    r = jax.nn.sigmoid(gi[:, :_FH] + gh[:, :_FH])
    z = jax.nn.sigmoid(gi[:, _FH:2 * _FH] + gh[:, _FH:2 * _FH])
    n = jnp.tanh(gi[:, 2 * _FH:] + r * (gh[:, 2 * _FH:] + bhn))
    h_new = (1.0 - z) * n + z * h
    return jnp.where(t0 + k < lfold, h_new, h)

  out_ref[...] = lax.fori_loop(0, _TC, step, h_ref[...])


@functools.cache
def _gru_chunk(t0):
  return pl.pallas_call(
      functools.partial(_gru_chunk_body, t0),
      in_specs=[
          pl.BlockSpec((_TC, _BQ, _FE), lambda: (0, 0, 0)),
          pl.BlockSpec((_BQ, _FH), lambda: (0, 0)),
          pl.BlockSpec((_BQ, _FH), lambda: (0, 0)),
          pl.BlockSpec((_FE, 3 * _FH), lambda: (0, 0)),
          pl.BlockSpec((_FH, 3 * _FH), lambda: (0, 0)),
          pl.BlockSpec((1, 3 * _FH), lambda: (0, 0)),
          pl.BlockSpec((1, _FH), lambda: (0, 0)),
      ],
      out_specs=pl.BlockSpec((_BQ, _FH), lambda: (0, 0)),
      out_shape=jax.ShapeDtypeStruct((_BQ, _FH), jnp.float32),
      scratch_shapes=[pltpu.VMEM((_TC, _BQ, 3 * _FH), jnp.float32)],
  )


def kernel(x, emb, w_ih, w_hh, b_ih, b_hh):
  idx_tm = x.T.reshape(_NCHUNK, _NW, _NGRP, _GRP)

  eye = jnp.eye(_F, dtype=jnp.float32)
  kr = lambda w: jnp.kron(eye, w)
  wi4 = jnp.concatenate(
      [kr(w_ih[g * HS:(g + 1) * HS, :].T) for g in range(3)],
      axis=1)
  wh4 = jnp.concatenate(
      [kr(w_hh[g * HS:(g + 1) * HS, :].T) for g in range(3)],
      axis=1)
  gib4 = jnp.concatenate([
      jnp.tile(b_ih[0:HS] + b_hh[0:HS], _F),
      jnp.tile(b_ih[HS:2 * HS] + b_hh[HS:2 * HS], _F),
      jnp.tile(b_ih[2 * HS:], _F),
  ])[None, :]
  bhn4 = jnp.tile(b_hh[2 * HS:], _F)[None, :]
  ones_t = jnp.ones((T, 1), jnp.float32)
  gg = jnp.arange(_F)
  jj = jnp.arange(_FH) // HS
  qmat = (gg[:, None] == jj[None, :]).astype(jnp.float32)

  lfold = _lfold(x.T, ones_t, qmat)
  gather = _sc_gather()
  h = jnp.zeros((_BQ, _FH), jnp.float32)
  for c in range(_NCHUNK):
    e3 = gather(emb, idx_tm[c])
    e4 = e3.reshape(_TC, _BQ, _FE)
    h = _gru_chunk(c * _TC)(e4, lfold, h, wi4, wh4, gib4, bhn4)
  return h.reshape(B, HS)


# speedup vs baseline: 1.0816x; 1.0816x over previous
import functools

import jax
import jax.numpy as jnp
from jax import lax
from jax.experimental import pallas as pl
from jax.experimental.pallas import tpu as pltpu
from jax.experimental.pallas import tpu_sc as plsc

VOCAB = 100000
ES = 32
HS = 64
B = 4096
T = 50

_CHUNKS = (10, 40)
_TC = 10

_NC = 2
_NS = 16
_NW = _NC * _NS
_GRP = 128
_FIRE = 10


def _sc_gather_body(ngrp, emb_hbm, idx_hbm, out_hbm, idx_v, rows_v, sem):
  wid = lax.axis_index("s") * _NC + lax.axis_index("c")
  pltpu.sync_copy(idx_hbm.at[wid], idx_v)

  def outer(o, carry):
    copies = []
    for j in range(_FIRE):
      cp = pltpu.async_copy(
          emb_hbm.at[idx_v.at[o * _FIRE + j]], rows_v.at[j], sem)
      copies.append(cp)
    for cp in copies:
      cp.wait()
    pltpu.sync_copy(rows_v, out_hbm.at[pl.ds(wid * ngrp + o * _FIRE, _FIRE)])
    return carry

  lax.fori_loop(0, ngrp // _FIRE, outer, 0)


@functools.cache
def _sc_gather(nt):
  ngrp = nt * B // _NW // _GRP
  return functools.partial(
      pl.kernel,
      out_type=jax.ShapeDtypeStruct((nt * B // _GRP, _GRP, ES), jnp.float32),
      mesh=plsc.VectorSubcoreMesh(core_axis_name="c", subcore_axis_name="s"),
      scratch_types=[
          pltpu.VMEM((ngrp, _GRP), jnp.int32),
          pltpu.VMEM((_FIRE, _GRP, ES), jnp.float32),
          pltpu.SemaphoreType.DMA,
      ],
      compiler_params=pltpu.CompilerParams(use_tc_tiling_on_sc=False),
  )(functools.partial(_sc_gather_body, ngrp))


_F = 4
_FH = _F * HS
_FE = _F * ES
_BQ = B // _F


def _lfold_body(xt_ref, ones_ref, q_ref, out_ref):
  ind = (xt_ref[...] != 0).astype(jnp.float32)
  lcol = lax.dot_general(ind, ones_ref[...], (((0,), (0,)), ((), ())),
                         preferred_element_type=jnp.float32)
  out_ref[...] = jnp.dot(lcol.reshape(_BQ, _F), q_ref[...],
                         preferred_element_type=jnp.float32).astype(jnp.int32)


def _lfold(xt, ones_t, qmat):
  return pl.pallas_call(
      _lfold_body,
      in_specs=[
          pl.BlockSpec((T, B), lambda: (0, 0)),
          pl.BlockSpec((T, 1), lambda: (0, 0)),
          pl.BlockSpec((_F, _FH), lambda: (0, 0)),
      ],
      out_specs=pl.BlockSpec((_BQ, _FH), lambda: (0, 0)),
      out_shape=jax.ShapeDtypeStruct((_BQ, _FH), jnp.int32),
  )(xt, ones_t, qmat)


def _gru_chunk_body(t0, nt, e_ref, lf_ref, h_ref, wi_ref, wh_ref, gib_ref,
                    bhn_ref, out_ref, gi_s, h_s):
  i = pl.program_id(0)

  @pl.when(i == 0)
  def _():
    h_s[...] = h_ref[...]

  e2d = e_ref[...].reshape(_TC * _BQ, _FE)
  gi_s[...] = (jnp.dot(e2d, wi_ref[...], preferred_element_type=jnp.float32)
               + gib_ref[...]).reshape(_TC, _BQ, 3 * _FH)
  wh = wh_ref[...]
  bhn = bhn_ref[...]
  lfold = lf_ref[...]
  tbase = t0 + i * _TC

  def step(k, h):
    gi = gi_s[k]
    gh = jnp.dot(h, wh, preferred_element_type=jnp.float32)
    r = jax.nn.sigmoid(gi[:, :_FH] + gh[:, :_FH])
    z = jax.nn.sigmoid(gi[:, _FH:2 * _FH] + gh[:, _FH:2 * _FH])
    n = jnp.tanh(gi[:, 2 * _FH:] + r * (gh[:, 2 * _FH:] + bhn))
    h_new = (1.0 - z) * n + z * h
    return jnp.where(tbase + k < lfold, h_new, h)

  h = lax.fori_loop(0, _TC, step, h_s[...])
  h_s[...] = h

  @pl.when(i == nt // _TC - 1)
  def _():
    out_ref[...] = h


@functools.cache
def _gru_chunk(t0, nt):
  return pl.pallas_call(
      functools.partial(_gru_chunk_body, t0, nt),
      grid=(nt // _TC,),
      in_specs=[
          pl.BlockSpec((_TC, _BQ, _FE), lambda i: (i, 0, 0)),
          pl.BlockSpec((_BQ, _FH), lambda i: (0, 0)),
          pl.BlockSpec((_BQ, _FH), lambda i: (0, 0)),
          pl.BlockSpec((_FE, 3 * _FH), lambda i: (0, 0)),
          pl.BlockSpec((_FH, 3 * _FH), lambda i: (0, 0)),
          pl.BlockSpec((1, 3 * _FH), lambda i: (0, 0)),
          pl.BlockSpec((1, _FH), lambda i: (0, 0)),
      ],
      out_specs=pl.BlockSpec((_BQ, _FH), lambda i: (0, 0)),
      out_shape=jax.ShapeDtypeStruct((_BQ, _FH), jnp.float32),
      scratch_shapes=[
          pltpu.VMEM((_TC, _BQ, 3 * _FH), jnp.float32),
          pltpu.VMEM((_BQ, _FH), jnp.float32),
      ],
      compiler_params=pltpu.CompilerParams(
          dimension_semantics=("arbitrary",),
      ),
  )


def kernel(x, emb, w_ih, w_hh, b_ih, b_hh):
  idx_tm = x.T.reshape(T * B // _GRP, _GRP)

  eye = jnp.eye(_F, dtype=jnp.float32)
  kr = lambda w: jnp.kron(eye, w)
  wi4 = jnp.concatenate(
      [kr(w_ih[g * HS:(g + 1) * HS, :].T) for g in range(3)],
      axis=1)
  wh4 = jnp.concatenate(
      [kr(w_hh[g * HS:(g + 1) * HS, :].T) for g in range(3)],
      axis=1)
  gib4 = jnp.concatenate([
      jnp.tile(b_ih[0:HS] + b_hh[0:HS], _F),
      jnp.tile(b_ih[HS:2 * HS] + b_hh[HS:2 * HS], _F),
      jnp.tile(b_ih[2 * HS:], _F),
  ])[None, :]
  bhn4 = jnp.tile(b_hh[2 * HS:], _F)[None, :]
  ones_t = jnp.ones((T, 1), jnp.float32)
  gg = jnp.arange(_F)
  jj = jnp.arange(_FH) // HS
  qmat = (gg[:, None] == jj[None, :]).astype(jnp.float32)

  lfold = _lfold(x.T, ones_t, qmat)
  h = jnp.zeros((_BQ, _FH), jnp.float32)
  t0 = 0
  for nt in _CHUNKS:
    g0 = t0 * B // _GRP
    idx3 = idx_tm[g0:g0 + nt * B // _GRP].reshape(
        _NW, nt * B // _NW // _GRP, _GRP)
    e3 = _sc_gather(nt)(emb, idx3)
    e4 = e3.reshape(nt, _BQ, _FE)
    h = _gru_chunk(t0, nt)(e4, lfold, h, wi4, wh4, gib4, bhn4)
    t0 += nt
  return h.reshape(B, HS)
